# baseline (speedup 1.0000x reference)
import functools

import jax
import jax.numpy as jnp
from jax import lax
from jax.experimental import pallas as pl
from jax.experimental.pallas import tpu as pltpu
from jax.experimental.pallas import tpu_sc as plsc

N = 10000
E = 320000
D = 128
B = 64

NC = 2
NS = 16
NW = NC * NS
CHUNK = 128
NRING = 3
NPIECE = 28
NCHUNK = NPIECE * NRING
EPT = NCHUNK * CHUNK
E_PAD = NW * EPT
NPAD = 10016
RPT = NPAD // NS

_mesh = plsc.VectorSubcoreMesh(
    core_axis_name="c", subcore_axis_name="s", num_cores=NC, num_subcores=NS)

_sc_params = pltpu.CompilerParams(use_tc_tiling_on_sc=False)

_f32 = jnp.float32


@functools.partial(
    pl.kernel,
    out_type=[jax.ShapeDtypeStruct((NPAD, 16), _f32),
              jax.ShapeDtypeStruct((NPAD, 16), _f32)],
    mesh=_mesh,
    scratch_types=[
        pltpu.VMEM((NCHUNK, CHUNK), jnp.int32),
        pltpu.VMEM((CHUNK, 16), _f32),
        pltpu.VMEM((RPT, 16), _f32),
        pltpu.VMEM_SHARED((NPAD, 16), _f32),
    ],
    compiler_params=_sc_params,
)
def _sc_degree(dst_hbm, out0, out1, dst_v, ones_v, zb_v, acc):
    c = lax.axis_index("c")
    s = lax.axis_index("s")
    wid = s * NC + c
    ones16 = jnp.ones((16,), _f32)
    zeros16 = jnp.zeros((16,), _f32)

    def fill_ones(i, _):
        ones_v[i] = ones16
        return 0
    lax.fori_loop(0, CHUNK, fill_ones, 0)

    def fill_zeros(i, _):
        zb_v[i] = zeros16
        return 0
    lax.fori_loop(0, RPT, fill_zeros, 0)

    pltpu.sync_copy(zb_v, acc.at[pl.ds(s * RPT, RPT)])
    pltpu.sync_copy(dst_hbm.at[wid], dst_v)
    plsc.subcore_barrier()

    def body(j, _):
        pltpu.sync_copy(ones_v, acc.at[dst_v.at[j]], add=True)
        return 0
    lax.fori_loop(0, NCHUNK, body, 0)

    plsc.subcore_barrier()

    @pl.when(c == 0)
    def _():
        pltpu.sync_copy(acc.at[pl.ds(s * RPT, RPT)], out0.at[pl.ds(s * RPT, RPT)])

    @pl.when(c == 1)
    def _():
        pltpu.sync_copy(acc.at[pl.ds(s * RPT, RPT)], out1.at[pl.ds(s * RPT, RPT)])


@functools.partial(
    pl.kernel,
    out_type=[jax.ShapeDtypeStruct((NPAD, D), _f32),
              jax.ShapeDtypeStruct((NPAD, D), _f32)],
    mesh=_mesh,
    scratch_types=[
        [pltpu.VMEM((NRING, CHUNK), jnp.int32)] * 2,
        [pltpu.VMEM((NRING, CHUNK), jnp.int32)] * 2,
        [pltpu.VMEM((CHUNK, D), _f32)] * NRING,
        [pltpu.SemaphoreType.DMA] * NRING,
        [pltpu.SemaphoreType.DMA] * 2,
        pltpu.VMEM_SHARED((NPAD, D), _f32),
    ],
    compiler_params=_sc_params,
)
def _sc_scatter(g_hbm, src_hbm, dst_hbm, out0, out1,
                sp, dp, rows, gsem, isem, acc):
    c = lax.axis_index("c")
    s = lax.axis_index("s")
    wid = s * NC + c
    zeros16 = jnp.zeros((16,), _f32)

    def zrow(r, _):
        def zcol(k, _2):
            rows[0][r, pl.ds(k * 16, 16)] = zeros16
            return 0
        lax.fori_loop(0, D // 16, zcol, 0)
        return 0
    lax.fori_loop(0, CHUNK, zrow, 0)

    for k in range(RPT // CHUNK):
        pltpu.sync_copy(rows[0], acc.at[pl.ds(s * RPT + k * CHUNK, CHUNK)])
    rem = RPT % CHUNK
    if rem:
        pltpu.sync_copy(rows[0].at[pl.ds(0, rem)],
                        acc.at[pl.ds(s * RPT + (RPT // CHUNK) * CHUNK, rem)])

    pltpu.sync_copy(src_hbm.at[wid, 0], sp[0])
    pltpu.sync_copy(dst_hbm.at[wid, 0], dp[0])
    plsc.subcore_barrier()

    for b in range(NRING):
        pltpu.async_copy(g_hbm.at[sp[0].at[b]], rows[b], gsem[b])
    pltpu.async_copy(src_hbm.at[wid, 1], sp[1], isem[1])
    pltpu.async_copy(dst_hbm.at[wid, 1], dp[1], isem[1])

    def phase(j, pp):
        p = 2 * j + pp
        spc, dpc = sp[pp], dp[pp]
        spn, dpn = sp[1 - pp], dp[1 - pp]

        @pl.when(p < NPIECE - 1)
        def _():
            pltpu.make_async_copy(src_hbm.at[wid, 0], spn, isem[1 - pp]).wait()
            pltpu.make_async_copy(dst_hbm.at[wid, 0], dpn, isem[1 - pp]).wait()

        for b in range(NRING):
            pltpu.make_async_copy(g_hbm.at[spc.at[b]], rows[b], gsem[b]).wait()
            pltpu.sync_copy(rows[b], acc.at[dpc.at[b]], add=True)

            @pl.when(p < NPIECE - 1)
            def _(b=b):
                pltpu.async_copy(g_hbm.at[spn.at[b]], rows[b], gsem[b])

        @pl.when(p < NPIECE - 2)
        def _():
            pltpu.async_copy(src_hbm.at[wid, p + 2], spc, isem[pp])
            pltpu.async_copy(dst_hbm.at[wid, p + 2], dpc, isem[pp])

    def body(j, _):
        phase(j, 0)
        phase(j, 1)
        return 0
    lax.fori_loop(0, NPIECE // 2, body, 0)

    plsc.subcore_barrier()

    @pl.when(c == 0)
    def _():
        pltpu.sync_copy(acc.at[pl.ds(s * RPT, RPT)], out0.at[pl.ds(s * RPT, RPT)])

    @pl.when(c == 1)
    def _():
        pltpu.sync_copy(acc.at[pl.ds(s * RPT, RPT)], out1.at[pl.ds(s * RPT, RPT)])


RB = 1000
_HI = jax.lax.Precision.HIGHEST


def _tc1_body(d0, d1, xr, w1, g1o, dvo):
    deg = 1.0 + d0[:, :1] + d1[:, :1]
    dv = jax.lax.rsqrt(deg)
    h = jnp.dot(xr[...], w1[...], preferred_element_type=_f32, precision=_HI)
    g1o[...] = h * dv
    dvo[...] = dv


def _tc2_body(a0, a1, g1r, dv, b1r, w2, g2o):
    z = (a0[...] + a1[...] + g1r[...]) * dv[...] + b1r[...]
    z = jnp.maximum(z, 0.0)
    g2o[...] = jnp.dot(z, w2[...], preferred_element_type=_f32,
                       precision=_HI) * dv[...]


def _tc3_body(a0, a1, g2r, dv, b2r, batch_r, out_ref, acc, cnt):
    i = pl.program_id(0)

    @pl.when(i == 0)
    def _():
        acc[...] = jnp.zeros_like(acc)
        cnt[...] = jnp.zeros_like(cnt)

    h = (a0[...] + a1[...] + g2r[...]) * dv[...] + b2r[...]
    bvec = batch_r[0]
    onehot = (lax.broadcasted_iota(jnp.int32, (B, RB), 0) == bvec).astype(_f32)
    acc[...] += jnp.dot(onehot, h, preferred_element_type=_f32, precision=_HI)
    cnt[...] += jnp.sum(onehot, axis=1, keepdims=True)

    @pl.when(i == pl.num_programs(0) - 1)
    def _():
        out_ref[...] = acc[...] / jnp.maximum(cnt[...], 1.0)


def _row_spec(w):
    return pl.BlockSpec((RB, w), lambda i: (i, 0))


def _const_spec(shape):
    return pl.BlockSpec(shape, lambda i: (0, 0))


def kernel(x, edge_index, batch, W1, b1, W2, b2):
    src = edge_index[0]
    dst = edge_index[1]
    pad = E_PAD - E
    src_p = jnp.concatenate([src, jnp.zeros((pad,), jnp.int32)])
    dst_p = jnp.concatenate([dst, jnp.full((pad,), N, jnp.int32)])
    dst_c = dst_p.reshape(NW, NCHUNK, CHUNK)
    src_q = src_p.reshape(NW, NPIECE, NRING, CHUNK)
    dst_q = dst_p.reshape(NW, NPIECE, NRING, CHUNK)
    b1r = b1.reshape(1, D)
    b2r = b2.reshape(1, D)
    batch_r = batch.reshape(N // RB, 1, RB)

    degp0, degp1 = _sc_degree(dst_c)

    g1, dinv = pl.pallas_call(
        _tc1_body,
        grid=(N // RB,),
        in_specs=[_row_spec(16), _row_spec(16), _row_spec(D), _const_spec((D, D))],
        out_specs=[_row_spec(D), _row_spec(1)],
        out_shape=[jax.ShapeDtypeStruct((N, D), _f32),
                   jax.ShapeDtypeStruct((N, 1), _f32)],
    )(degp0, degp1, x, W1)

    a0, a1 = _sc_scatter(g1, src_q, dst_q)

    g2 = pl.pallas_call(
        _tc2_body,
        grid=(N // RB,),
        in_specs=[_row_spec(D), _row_spec(D), _row_spec(D), _row_spec(1),
                  _const_spec((1, D)), _const_spec((D, D))],
        out_specs=_row_spec(D),
        out_shape=jax.ShapeDtypeStruct((N, D), _f32),
    )(a0, a1, g1, dinv, b1r, W2)

    c0, c1 = _sc_scatter(g2, src_q, dst_q)

    out = pl.pallas_call(
        _tc3_body,
        grid=(N // RB,),
        in_specs=[_row_spec(D), _row_spec(D), _row_spec(D), _row_spec(1),
                  _const_spec((1, D)),
                  pl.BlockSpec((1, 1, RB), lambda i: (i, 0, 0))],
        out_specs=_const_spec((B, D)),
        out_shape=jax.ShapeDtypeStruct((B, D), _f32),
        scratch_shapes=[pltpu.VMEM((B, D), _f32), pltpu.VMEM((B, D), _f32)],
    )(c0, c1, g2, dinv, b2r, batch_r)

    return out


# speedup vs baseline: 1.8798x; 1.8798x over previous
import functools

import jax
import jax.numpy as jnp
from jax import lax
from jax.experimental import pallas as pl
from jax.experimental.pallas import tpu as pltpu
from jax.experimental.pallas import tpu_sc as plsc

N = 10000
E = 320000
D = 128
B = 64

NC = 2
NS = 16
NW = NC * NS
CHUNK = 128
NCHUNK = 80
TOTCH = NW * NCHUNK
NCH0 = 48
NCH1 = 112
NCMAX = max(NCH0, NCH1)
NRING = 1
E_PAD = TOTCH * CHUNK
NPAD = 10112
RPT = NPAD // NS

_mesh = plsc.VectorSubcoreMesh(
    core_axis_name="c", subcore_axis_name="s", num_cores=NC, num_subcores=NS)

_sc_params = pltpu.CompilerParams(use_tc_tiling_on_sc=False)

_f32 = jnp.float32


@functools.partial(
    pl.kernel,
    out_type=[jax.ShapeDtypeStruct((NPAD, 16), _f32),
              jax.ShapeDtypeStruct((NPAD, 16), _f32)],
    mesh=_mesh,
    scratch_types=[
        pltpu.VMEM((NCHUNK, CHUNK), jnp.int32),
        pltpu.VMEM((CHUNK, 16), _f32),
        pltpu.VMEM((RPT, 16), _f32),
        pltpu.VMEM_SHARED((NPAD, 16), _f32),
    ],
    compiler_params=_sc_params,
)
def _sc_degree(dst_hbm, out0, out1, dst_v, ones_v, zb_v, acc):
    c = lax.axis_index("c")
    s = lax.axis_index("s")
    wid = s * NC + c
    ones16 = jnp.ones((16,), _f32)
    zeros16 = jnp.zeros((16,), _f32)

    def fill_ones(i, _):
        ones_v[i] = ones16
        return 0
    lax.fori_loop(0, CHUNK, fill_ones, 0)

    def fill_zeros(i, _):
        zb_v[i] = zeros16
        return 0
    lax.fori_loop(0, RPT, fill_zeros, 0)

    pltpu.sync_copy(zb_v, acc.at[pl.ds(s * RPT, RPT)])
    pltpu.sync_copy(dst_hbm.at[wid], dst_v)
    plsc.subcore_barrier()

    def body(j, _):
        pltpu.sync_copy(ones_v, acc.at[dst_v.at[j]], add=True)
        return 0
    lax.fori_loop(0, NCHUNK, body, 0)

    plsc.subcore_barrier()

    @pl.when(c == 0)
    def _():
        pltpu.sync_copy(acc.at[pl.ds(s * RPT, RPT)], out0.at[pl.ds(s * RPT, RPT)])

    @pl.when(c == 1)
    def _():
        pltpu.sync_copy(acc.at[pl.ds(s * RPT, RPT)], out1.at[pl.ds(s * RPT, RPT)])


@functools.partial(
    pl.kernel,
    out_type=[jax.ShapeDtypeStruct((NPAD, D), _f32),
              jax.ShapeDtypeStruct((NPAD, D), _f32)],
    mesh=_mesh,
    scratch_types=[
        pltpu.VMEM((NCMAX, CHUNK), jnp.int32),
        pltpu.VMEM((NCMAX, CHUNK), jnp.int32),
        [pltpu.VMEM((CHUNK, D), _f32)] * NRING,
        [pltpu.SemaphoreType.DMA] * NRING,
        [pltpu.SemaphoreType.DMA] * NRING,
        pltpu.VMEM_SHARED((NPAD, D), _f32),
    ],
    compiler_params=_sc_params,
)
def _sc_scatter(g_hbm, src_hbm, dst_hbm, out0, out1,
                src_v, dst_v, rows, gsem, ssem, acc):
    c = lax.axis_index("c")
    s = lax.axis_index("s")
    zeros16 = jnp.zeros((16,), _f32)

    def zrow(r, _):
        def zcol(k, _2):
            rows[0][r, pl.ds(k * 16, 16)] = zeros16
            return 0
        lax.fori_loop(0, D // 16, zcol, 0)
        return 0
    lax.fori_loop(0, CHUNK, zrow, 0)

    for k in range(RPT // CHUNK):
        pltpu.sync_copy(rows[0], acc.at[pl.ds(s * RPT + k * CHUNK, CHUNK)])
    rem = RPT % CHUNK
    pltpu.sync_copy(rows[0].at[pl.ds(0, rem)],
                    acc.at[pl.ds(s * RPT + (RPT // CHUNK) * CHUNK, rem)])

    def body(j, _):
        pltpu.async_copy(g_hbm.at[src_v.at[j]], rows[0], gsem[0]).wait()
        pltpu.sync_copy(rows[0], acc.at[dst_v.at[j]], add=True)
        return 0

    @pl.when(c == 0)
    def _():
        base = s * NCH0
        pltpu.sync_copy(src_hbm.at[pl.ds(base, NCH0)], src_v.at[pl.ds(0, NCH0)])
        pltpu.sync_copy(dst_hbm.at[pl.ds(base, NCH0)], dst_v.at[pl.ds(0, NCH0)])
        plsc.subcore_barrier()
        lax.fori_loop(0, NCH0, body, 0)

    @pl.when(c == 1)
    def _():
        base = NS * NCH0 + s * NCH1
        pltpu.sync_copy(src_hbm.at[pl.ds(base, NCH1)], src_v.at[pl.ds(0, NCH1)])
        pltpu.sync_copy(dst_hbm.at[pl.ds(base, NCH1)], dst_v.at[pl.ds(0, NCH1)])
        plsc.subcore_barrier()
        lax.fori_loop(0, NCH1, body, 0)

    plsc.subcore_barrier()

    @pl.when(c == 0)
    def _():
        pltpu.sync_copy(acc.at[pl.ds(s * RPT, RPT)], out0.at[pl.ds(s * RPT, RPT)])

    @pl.when(c == 1)
    def _():
        pltpu.sync_copy(acc.at[pl.ds(s * RPT, RPT)], out1.at[pl.ds(s * RPT, RPT)])


RB = 1000
_HI = jax.lax.Precision.HIGHEST


def _tc1_body(d0, d1, xr, w1, g1o, dvo):
    deg = 1.0 + d0[:, :1] + d1[:, :1]
    dv = jax.lax.rsqrt(deg)
    h = jnp.dot(xr[...], w1[...], preferred_element_type=_f32, precision=_HI)
    g1o[...] = h * dv
    dvo[...] = dv


def _tc2_body(a0, a1, g1r, dv, b1r, w2, g2o):
    z = (a0[...] + a1[...] + g1r[...]) * dv[...] + b1r[...]
    z = jnp.maximum(z, 0.0)
    g2o[...] = jnp.dot(z, w2[...], preferred_element_type=_f32,
                       precision=_HI) * dv[...]


def _tc3_body(a0, a1, g2r, dv, b2r, batch_r, out_ref, acc, cnt):
    i = pl.program_id(0)

    @pl.when(i == 0)
    def _():
        acc[...] = jnp.zeros_like(acc)
        cnt[...] = jnp.zeros_like(cnt)

    h = (a0[...] + a1[...] + g2r[...]) * dv[...] + b2r[...]
    bvec = batch_r[0]
    onehot = (lax.broadcasted_iota(jnp.int32, (B, RB), 0) == bvec).astype(_f32)
    acc[...] += jnp.dot(onehot, h, preferred_element_type=_f32, precision=_HI)
    cnt[...] += jnp.sum(onehot, axis=1, keepdims=True)

    @pl.when(i == pl.num_programs(0) - 1)
    def _():
        out_ref[...] = acc[...] / jnp.maximum(cnt[...], 1.0)


def _row_spec(w):
    return pl.BlockSpec((RB, w), lambda i: (i, 0))


def _const_spec(shape):
    return pl.BlockSpec(shape, lambda i: (0, 0))


def kernel(x, edge_index, batch, W1, b1, W2, b2):
    src = edge_index[0]
    dst = edge_index[1]
    pad = E_PAD - E
    src_p = jnp.concatenate([src, jnp.zeros((pad,), jnp.int32)])
    dst_p = jnp.concatenate([dst, jnp.full((pad,), N, jnp.int32)])
    src_f = src_p.reshape(TOTCH, CHUNK)
    dst_f = dst_p.reshape(TOTCH, CHUNK)
    dst_c = dst_p.reshape(NW, NCHUNK, CHUNK)
    b1r = b1.reshape(1, D)
    b2r = b2.reshape(1, D)
    batch_r = batch.reshape(N // RB, 1, RB)

    degp0, degp1 = _sc_degree(dst_c)

    g1, dinv = pl.pallas_call(
        _tc1_body,
        grid=(N // RB,),
        in_specs=[_row_spec(16), _row_spec(16), _row_spec(D), _const_spec((D, D))],
        out_specs=[_row_spec(D), _row_spec(1)],
        out_shape=[jax.ShapeDtypeStruct((N, D), _f32),
                   jax.ShapeDtypeStruct((N, 1), _f32)],
    )(degp0, degp1, x, W1)

    a0, a1 = _sc_scatter(g1, src_f, dst_f)

    g2 = pl.pallas_call(
        _tc2_body,
        grid=(N // RB,),
        in_specs=[_row_spec(D), _row_spec(D), _row_spec(D), _row_spec(1),
                  _const_spec((1, D)), _const_spec((D, D))],
        out_specs=_row_spec(D),
        out_shape=jax.ShapeDtypeStruct((N, D), _f32),
    )(a0, a1, g1, dinv, b1r, W2)

    c0, c1 = _sc_scatter(g2, src_f, dst_f)

    out = pl.pallas_call(
        _tc3_body,
        grid=(N // RB,),
        in_specs=[_row_spec(D), _row_spec(D), _row_spec(D), _row_spec(1),
                  _const_spec((1, D)),
                  pl.BlockSpec((1, 1, RB), lambda i: (i, 0, 0))],
        out_specs=_const_spec((B, D)),
        out_shape=jax.ShapeDtypeStruct((B, D), _f32),
        scratch_shapes=[pltpu.VMEM((B, D), _f32), pltpu.VMEM((B, D), _f32)],
    )(c0, c1, g2, dinv, b2r, batch_r)

    return out


# speedup vs baseline: 2.3017x; 1.2244x over previous
import functools

import jax
import jax.numpy as jnp
from jax import lax
from jax.experimental import pallas as pl
from jax.experimental.pallas import tpu as pltpu
from jax.experimental.pallas import tpu_sc as plsc

N = 10000
E = 320000
D = 128
B = 64

NC = 2
NS = 16
NW = NC * NS
CHUNK = 128
NCHUNK = 80
TOTCH = NW * NCHUNK
NCH0 = 112
NCH1 = 48
NCMAX = max(NCH0, NCH1)
NRING = 1
E_PAD = TOTCH * CHUNK
NPAD = 10112
RPT = NPAD // NS

_mesh = plsc.VectorSubcoreMesh(
    core_axis_name="c", subcore_axis_name="s", num_cores=NC, num_subcores=NS)

_sc_params = pltpu.CompilerParams(use_tc_tiling_on_sc=False)

_f32 = jnp.float32


@functools.partial(
    pl.kernel,
    out_type=[jax.ShapeDtypeStruct((NPAD, 16), _f32),
              jax.ShapeDtypeStruct((NPAD, 16), _f32)],
    mesh=_mesh,
    scratch_types=[
        pltpu.VMEM((NCHUNK, CHUNK), jnp.int32),
        pltpu.VMEM((CHUNK, 16), _f32),
        pltpu.VMEM((RPT, 16), _f32),
        pltpu.VMEM_SHARED((NPAD, 16), _f32),
    ],
    compiler_params=_sc_params,
)
def _sc_degree(dst_hbm, out0, out1, dst_v, ones_v, zb_v, acc):
    c = lax.axis_index("c")
    s = lax.axis_index("s")
    wid = s * NC + c
    ones16 = jnp.ones((16,), _f32)
    zeros16 = jnp.zeros((16,), _f32)

    def fill_ones(i, _):
        ones_v[i] = ones16
        return 0
    lax.fori_loop(0, CHUNK, fill_ones, 0)

    def fill_zeros(i, _):
        zb_v[i] = zeros16
        return 0
    lax.fori_loop(0, RPT, fill_zeros, 0)

    pltpu.sync_copy(zb_v, acc.at[pl.ds(s * RPT, RPT)])
    pltpu.sync_copy(dst_hbm.at[wid], dst_v)
    plsc.subcore_barrier()

    def body(j, _):
        pltpu.sync_copy(ones_v, acc.at[dst_v.at[j]], add=True)
        return 0
    lax.fori_loop(0, NCHUNK, body, 0)

    plsc.subcore_barrier()

    @pl.when(c == 0)
    def _():
        pltpu.sync_copy(acc.at[pl.ds(s * RPT, RPT)], out0.at[pl.ds(s * RPT, RPT)])

    @pl.when(c == 1)
    def _():
        pltpu.sync_copy(acc.at[pl.ds(s * RPT, RPT)], out1.at[pl.ds(s * RPT, RPT)])


@functools.partial(
    pl.kernel,
    out_type=[jax.ShapeDtypeStruct((NPAD, D), _f32),
              jax.ShapeDtypeStruct((NPAD, D), _f32)],
    mesh=_mesh,
    scratch_types=[
        pltpu.VMEM((NCMAX, CHUNK), jnp.int32),
        pltpu.VMEM((NCMAX, CHUNK), jnp.int32),
        [pltpu.VMEM((CHUNK, D), _f32)] * NRING,
        [pltpu.SemaphoreType.DMA] * NRING,
        [pltpu.SemaphoreType.DMA] * NRING,
        pltpu.VMEM_SHARED((NPAD, D), _f32),
    ],
    compiler_params=_sc_params,
)
def _sc_scatter(g_hbm, src_hbm, dst_hbm, out0, out1,
                src_v, dst_v, rows, gsem, ssem, acc):
    c = lax.axis_index("c")
    s = lax.axis_index("s")
    zeros16 = jnp.zeros((16,), _f32)

    def zrow(r, _):
        def zcol(k, _2):
            rows[0][r, pl.ds(k * 16, 16)] = zeros16
            return 0
        lax.fori_loop(0, D // 16, zcol, 0)
        return 0
    lax.fori_loop(0, CHUNK, zrow, 0)

    for k in range(RPT // CHUNK):
        pltpu.sync_copy(rows[0], acc.at[pl.ds(s * RPT + k * CHUNK, CHUNK)])
    rem = RPT % CHUNK
    pltpu.sync_copy(rows[0].at[pl.ds(0, rem)],
                    acc.at[pl.ds(s * RPT + (RPT // CHUNK) * CHUNK, rem)])

    def body(j, _):
        pltpu.async_copy(g_hbm.at[src_v.at[j]], rows[0], gsem[0]).wait()
        pltpu.sync_copy(rows[0], acc.at[dst_v.at[j]], add=True)
        return 0

    @pl.when(c == 0)
    def _():
        base = s * NCH0
        pltpu.sync_copy(src_hbm.at[pl.ds(base, NCH0)], src_v.at[pl.ds(0, NCH0)])
        pltpu.sync_copy(dst_hbm.at[pl.ds(base, NCH0)], dst_v.at[pl.ds(0, NCH0)])
        plsc.subcore_barrier()
        lax.fori_loop(0, NCH0, body, 0)

    @pl.when(c == 1)
    def _():
        base = NS * NCH0 + s * NCH1
        pltpu.sync_copy(src_hbm.at[pl.ds(base, NCH1)], src_v.at[pl.ds(0, NCH1)])
        pltpu.sync_copy(dst_hbm.at[pl.ds(base, NCH1)], dst_v.at[pl.ds(0, NCH1)])
        plsc.subcore_barrier()
        lax.fori_loop(0, NCH1, body, 0)

    plsc.subcore_barrier()

    @pl.when(c == 0)
    def _():
        pltpu.sync_copy(acc.at[pl.ds(s * RPT, RPT)], out0.at[pl.ds(s * RPT, RPT)])

    @pl.when(c == 1)
    def _():
        pltpu.sync_copy(acc.at[pl.ds(s * RPT, RPT)], out1.at[pl.ds(s * RPT, RPT)])


RB = 1000
_HI = jax.lax.Precision.HIGHEST


def _tc1_body(d0, d1, xr, w1, g1o, dvo):
    deg = 1.0 + d0[:, :1] + d1[:, :1]
    dv = jax.lax.rsqrt(deg)
    h = jnp.dot(xr[...], w1[...], preferred_element_type=_f32, precision=_HI)
    g1o[...] = h * dv
    dvo[...] = dv


def _tc2_body(a0, a1, g1r, dv, b1r, w2, g2o):
    z = (a0[...] + a1[...] + g1r[...]) * dv[...] + b1r[...]
    z = jnp.maximum(z, 0.0)
    g2o[...] = jnp.dot(z, w2[...], preferred_element_type=_f32,
                       precision=_HI) * dv[...]


def _tc3_body(a0, a1, g2r, dv, b2r, batch_r, out_ref, acc, cnt):
    i = pl.program_id(0)

    @pl.when(i == 0)
    def _():
        acc[...] = jnp.zeros_like(acc)
        cnt[...] = jnp.zeros_like(cnt)

    h = (a0[...] + a1[...] + g2r[...]) * dv[...] + b2r[...]
    bvec = batch_r[0]
    onehot = (lax.broadcasted_iota(jnp.int32, (B, RB), 0) == bvec).astype(_f32)
    acc[...] += jnp.dot(onehot, h, preferred_element_type=_f32, precision=_HI)
    cnt[...] += jnp.sum(onehot, axis=1, keepdims=True)

    @pl.when(i == pl.num_programs(0) - 1)
    def _():
        out_ref[...] = acc[...] / jnp.maximum(cnt[...], 1.0)


def _row_spec(w):
    return pl.BlockSpec((RB, w), lambda i: (i, 0))


def _const_spec(shape):
    return pl.BlockSpec(shape, lambda i: (0, 0))


def kernel(x, edge_index, batch, W1, b1, W2, b2):
    src = edge_index[0]
    dst = edge_index[1]
    pad = E_PAD - E
    src_p = jnp.concatenate([src, jnp.zeros((pad,), jnp.int32)])
    dst_p = jnp.concatenate([dst, jnp.full((pad,), N, jnp.int32)])
    src_f = src_p.reshape(TOTCH, CHUNK)
    dst_f = dst_p.reshape(TOTCH, CHUNK)
    dst_c = dst_p.reshape(NW, NCHUNK, CHUNK)
    b1r = b1.reshape(1, D)
    b2r = b2.reshape(1, D)
    batch_r = batch.reshape(N // RB, 1, RB)

    degp0, degp1 = _sc_degree(dst_c)

    g1, dinv = pl.pallas_call(
        _tc1_body,
        grid=(N // RB,),
        in_specs=[_row_spec(16), _row_spec(16), _row_spec(D), _const_spec((D, D))],
        out_specs=[_row_spec(D), _row_spec(1)],
        out_shape=[jax.ShapeDtypeStruct((N, D), _f32),
                   jax.ShapeDtypeStruct((N, 1), _f32)],
    )(degp0, degp1, x, W1)

    a0, a1 = _sc_scatter(g1, src_f, dst_f)

    g2 = pl.pallas_call(
        _tc2_body,
        grid=(N // RB,),
        in_specs=[_row_spec(D), _row_spec(D), _row_spec(D), _row_spec(1),
                  _const_spec((1, D)), _const_spec((D, D))],
        out_specs=_row_spec(D),
        out_shape=jax.ShapeDtypeStruct((N, D), _f32),
    )(a0, a1, g1, dinv, b1r, W2)

    c0, c1 = _sc_scatter(g2, src_f, dst_f)

    out = pl.pallas_call(
        _tc3_body,
        grid=(N // RB,),
        in_specs=[_row_spec(D), _row_spec(D), _row_spec(D), _row_spec(1),
                  _const_spec((1, D)),
                  pl.BlockSpec((1, 1, RB), lambda i: (i, 0, 0))],
        out_specs=_const_spec((B, D)),
        out_shape=jax.ShapeDtypeStruct((B, D), _f32),
        scratch_shapes=[pltpu.VMEM((B, D), _f32), pltpu.VMEM((B, D), _f32)],
    )(c0, c1, g2, dinv, b2r, batch_r)

    return out


# speedup vs baseline: 2.4599x; 1.0687x over previous
import functools

import jax
import jax.numpy as jnp
from jax import lax
from jax.experimental import pallas as pl
from jax.experimental.pallas import tpu as pltpu
from jax.experimental.pallas import tpu_sc as plsc

N = 10000
E = 320000
D = 128
B = 64

NC = 2
NS = 16
NW = NC * NS
CHUNK = 128
NCHUNK = 80
TOTCH = NW * NCHUNK
NCH0 = 128
NCH1 = 32
NCMAX = max(NCH0, NCH1)
NRING = 1
E_PAD = TOTCH * CHUNK
NPAD = 10112
RPT = NPAD // NS

_mesh = plsc.VectorSubcoreMesh(
    core_axis_name="c", subcore_axis_name="s", num_cores=NC, num_subcores=NS)

_sc_params = pltpu.CompilerParams(use_tc_tiling_on_sc=False)

_f32 = jnp.float32


@functools.partial(
    pl.kernel,
    out_type=[jax.ShapeDtypeStruct((NPAD, 16), _f32),
              jax.ShapeDtypeStruct((NPAD, 16), _f32)],
    mesh=_mesh,
    scratch_types=[
        pltpu.VMEM((NCHUNK, CHUNK), jnp.int32),
        pltpu.VMEM((CHUNK, 16), _f32),
        pltpu.VMEM((RPT, 16), _f32),
        pltpu.VMEM_SHARED((NPAD, 16), _f32),
    ],
    compiler_params=_sc_params,
)
def _sc_degree(dst_hbm, out0, out1, dst_v, ones_v, zb_v, acc):
    c = lax.axis_index("c")
    s = lax.axis_index("s")
    wid = s * NC + c
    ones16 = jnp.ones((16,), _f32)
    zeros16 = jnp.zeros((16,), _f32)

    def fill_ones(i, _):
        ones_v[i] = ones16
        return 0
    lax.fori_loop(0, CHUNK, fill_ones, 0)

    def fill_zeros(i, _):
        zb_v[i] = zeros16
        return 0
    lax.fori_loop(0, RPT, fill_zeros, 0)

    pltpu.sync_copy(zb_v, acc.at[pl.ds(s * RPT, RPT)])
    pltpu.sync_copy(dst_hbm.at[wid], dst_v)
    plsc.subcore_barrier()

    def body(j, _):
        pltpu.sync_copy(ones_v, acc.at[dst_v.at[j]], add=True)
        return 0
    lax.fori_loop(0, NCHUNK, body, 0)

    plsc.subcore_barrier()

    @pl.when(c == 0)
    def _():
        pltpu.sync_copy(acc.at[pl.ds(s * RPT, RPT)], out0.at[pl.ds(s * RPT, RPT)])

    @pl.when(c == 1)
    def _():
        pltpu.sync_copy(acc.at[pl.ds(s * RPT, RPT)], out1.at[pl.ds(s * RPT, RPT)])


@functools.partial(
    pl.kernel,
    out_type=[jax.ShapeDtypeStruct((NPAD, D), _f32),
              jax.ShapeDtypeStruct((NPAD, D), _f32)],
    mesh=_mesh,
    scratch_types=[
        pltpu.VMEM((NCMAX, CHUNK), jnp.int32),
        pltpu.VMEM((NCMAX, CHUNK), jnp.int32),
        [pltpu.VMEM((CHUNK, D), _f32)] * NRING,
        [pltpu.SemaphoreType.DMA] * NRING,
        [pltpu.SemaphoreType.DMA] * NRING,
        pltpu.VMEM_SHARED((NPAD, D), _f32),
    ],
    compiler_params=_sc_params,
)
def _sc_scatter(g_hbm, src_hbm, dst_hbm, out0, out1,
                src_v, dst_v, rows, gsem, ssem, acc):
    c = lax.axis_index("c")
    s = lax.axis_index("s")
    zeros16 = jnp.zeros((16,), _f32)

    def zrow(r, _):
        def zcol(k, _2):
            rows[0][r, pl.ds(k * 16, 16)] = zeros16
            return 0
        lax.fori_loop(0, D // 16, zcol, 0)
        return 0
    lax.fori_loop(0, CHUNK, zrow, 0)

    for k in range(RPT // CHUNK):
        pltpu.sync_copy(rows[0], acc.at[pl.ds(s * RPT + k * CHUNK, CHUNK)])
    rem = RPT % CHUNK
    pltpu.sync_copy(rows[0].at[pl.ds(0, rem)],
                    acc.at[pl.ds(s * RPT + (RPT // CHUNK) * CHUNK, rem)])

    def body(j, _):
        pltpu.async_copy(g_hbm.at[src_v.at[j]], rows[0], gsem[0]).wait()
        pltpu.sync_copy(rows[0], acc.at[dst_v.at[j]], add=True)
        return 0

    @pl.when(c == 0)
    def _():
        base = s * NCH0
        pltpu.sync_copy(src_hbm.at[pl.ds(base, NCH0)], src_v.at[pl.ds(0, NCH0)])
        pltpu.sync_copy(dst_hbm.at[pl.ds(base, NCH0)], dst_v.at[pl.ds(0, NCH0)])
        plsc.subcore_barrier()
        lax.fori_loop(0, NCH0, body, 0)

    @pl.when(c == 1)
    def _():
        base = NS * NCH0 + s * NCH1
        pltpu.sync_copy(src_hbm.at[pl.ds(base, NCH1)], src_v.at[pl.ds(0, NCH1)])
        pltpu.sync_copy(dst_hbm.at[pl.ds(base, NCH1)], dst_v.at[pl.ds(0, NCH1)])
        plsc.subcore_barrier()
        lax.fori_loop(0, NCH1, body, 0)

    plsc.subcore_barrier()

    @pl.when(c == 0)
    def _():
        pltpu.sync_copy(acc.at[pl.ds(s * RPT, RPT)], out0.at[pl.ds(s * RPT, RPT)])

    @pl.when(c == 1)
    def _():
        pltpu.sync_copy(acc.at[pl.ds(s * RPT, RPT)], out1.at[pl.ds(s * RPT, RPT)])


RB = 1000
_HI = jax.lax.Precision.HIGHEST


def _tc1_body(d0, d1, xr, w1, g1o, dvo):
    deg = 1.0 + d0[:, :1] + d1[:, :1]
    dv = jax.lax.rsqrt(deg)
    h = jnp.dot(xr[...], w1[...], preferred_element_type=_f32, precision=_HI)
    g1o[...] = h * dv
    dvo[...] = dv


def _tc2_body(a0, a1, g1r, dv, b1r, w2, g2o):
    z = (a0[...] + a1[...] + g1r[...]) * dv[...] + b1r[...]
    z = jnp.maximum(z, 0.0)
    g2o[...] = jnp.dot(z, w2[...], preferred_element_type=_f32,
                       precision=_HI) * dv[...]


def _tc3_body(a0, a1, g2r, dv, b2r, batch_r, out_ref, acc, cnt):
    i = pl.program_id(0)

    @pl.when(i == 0)
    def _():
        acc[...] = jnp.zeros_like(acc)
        cnt[...] = jnp.zeros_like(cnt)

    h = (a0[...] + a1[...] + g2r[...]) * dv[...] + b2r[...]
    bvec = batch_r[0]
    onehot = (lax.broadcasted_iota(jnp.int32, (B, RB), 0) == bvec).astype(_f32)
    acc[...] += jnp.dot(onehot, h, preferred_element_type=_f32, precision=_HI)
    cnt[...] += jnp.sum(onehot, axis=1, keepdims=True)

    @pl.when(i == pl.num_programs(0) - 1)
    def _():
        out_ref[...] = acc[...] / jnp.maximum(cnt[...], 1.0)


def _row_spec(w):
    return pl.BlockSpec((RB, w), lambda i: (i, 0))


def _const_spec(shape):
    return pl.BlockSpec(shape, lambda i: (0, 0))


def kernel(x, edge_index, batch, W1, b1, W2, b2):
    src = edge_index[0]
    dst = edge_index[1]
    pad = E_PAD - E
    src_p = jnp.concatenate([src, jnp.zeros((pad,), jnp.int32)])
    dst_p = jnp.concatenate([dst, jnp.full((pad,), N, jnp.int32)])
    src_f = src_p.reshape(TOTCH, CHUNK)
    dst_f = dst_p.reshape(TOTCH, CHUNK)
    dst_c = dst_p.reshape(NW, NCHUNK, CHUNK)
    b1r = b1.reshape(1, D)
    b2r = b2.reshape(1, D)
    batch_r = batch.reshape(N // RB, 1, RB)

    degp0, degp1 = _sc_degree(dst_c)

    g1, dinv = pl.pallas_call(
        _tc1_body,
        grid=(N // RB,),
        in_specs=[_row_spec(16), _row_spec(16), _row_spec(D), _const_spec((D, D))],
        out_specs=[_row_spec(D), _row_spec(1)],
        out_shape=[jax.ShapeDtypeStruct((N, D), _f32),
                   jax.ShapeDtypeStruct((N, 1), _f32)],
    )(degp0, degp1, x, W1)

    a0, a1 = _sc_scatter(g1, src_f, dst_f)

    g2 = pl.pallas_call(
        _tc2_body,
        grid=(N // RB,),
        in_specs=[_row_spec(D), _row_spec(D), _row_spec(D), _row_spec(1),
                  _const_spec((1, D)), _const_spec((D, D))],
        out_specs=_row_spec(D),
        out_shape=jax.ShapeDtypeStruct((N, D), _f32),
    )(a0, a1, g1, dinv, b1r, W2)

    c0, c1 = _sc_scatter(g2, src_f, dst_f)

    out = pl.pallas_call(
        _tc3_body,
        grid=(N // RB,),
        in_specs=[_row_spec(D), _row_spec(D), _row_spec(D), _row_spec(1),
                  _const_spec((1, D)),
                  pl.BlockSpec((1, 1, RB), lambda i: (i, 0, 0))],
        out_specs=_const_spec((B, D)),
        out_shape=jax.ShapeDtypeStruct((B, D), _f32),
        scratch_shapes=[pltpu.VMEM((B, D), _f32), pltpu.VMEM((B, D), _f32)],
    )(c0, c1, g2, dinv, b2r, batch_r)

    return out


# speedup vs baseline: 2.4602x; 1.0001x over previous
import functools

import jax
import jax.numpy as jnp
from jax import lax
from jax.experimental import pallas as pl
from jax.experimental.pallas import tpu as pltpu
from jax.experimental.pallas import tpu_sc as plsc

N = 10000
E = 320000
D = 128
B = 64

NC = 2
NS = 16
NW = NC * NS
CHUNK = 128
NCHUNK = 80
TOTCH = NW * NCHUNK
NCH0 = 128
NCH1 = 32
NCMAX = max(NCH0, NCH1)
NRING = 1
E_PAD = TOTCH * CHUNK
NPAD = 10112
RPT = NPAD // NS

_mesh = plsc.VectorSubcoreMesh(
    core_axis_name="c", subcore_axis_name="s", num_cores=NC, num_subcores=NS)

_sc_params = pltpu.CompilerParams(use_tc_tiling_on_sc=False)

_f32 = jnp.float32


@functools.partial(
    pl.kernel,
    out_type=[jax.ShapeDtypeStruct((NPAD, 16), _f32),
              jax.ShapeDtypeStruct((NPAD, 16), _f32)],
    mesh=_mesh,
    scratch_types=[
        pltpu.VMEM((NCHUNK, CHUNK), jnp.int32),
        pltpu.VMEM((CHUNK, 16), _f32),
        pltpu.VMEM((RPT, 16), _f32),
        pltpu.VMEM_SHARED((NPAD, 16), _f32),
    ],
    compiler_params=_sc_params,
)
def _sc_degree(dst_hbm, out0, out1, dst_v, ones_v, zb_v, acc):
    c = lax.axis_index("c")
    s = lax.axis_index("s")
    wid = s * NC + c
    ones16 = jnp.ones((16,), _f32)
    zeros16 = jnp.zeros((16,), _f32)

    def fill_ones(i, _):
        ones_v[i] = ones16
        return 0
    lax.fori_loop(0, CHUNK, fill_ones, 0)

    def fill_zeros(i, _):
        zb_v[i] = zeros16
        return 0
    lax.fori_loop(0, RPT, fill_zeros, 0)

    pltpu.sync_copy(zb_v, acc.at[pl.ds(s * RPT, RPT)])
    pltpu.sync_copy(dst_hbm.at[wid], dst_v)
    plsc.subcore_barrier()

    def body(j, _):
        pltpu.sync_copy(ones_v, acc.at[dst_v.at[j]], add=True)
        return 0
    lax.fori_loop(0, NCHUNK, body, 0)

    plsc.subcore_barrier()

    @pl.when(c == 0)
    def _():
        pltpu.sync_copy(acc.at[pl.ds(s * RPT, RPT)], out0.at[pl.ds(s * RPT, RPT)])

    @pl.when(c == 1)
    def _():
        pltpu.sync_copy(acc.at[pl.ds(s * RPT, RPT)], out1.at[pl.ds(s * RPT, RPT)])


@functools.partial(
    pl.kernel,
    out_type=[jax.ShapeDtypeStruct((NPAD, D), _f32),
              jax.ShapeDtypeStruct((NPAD, D), _f32)],
    mesh=_mesh,
    scratch_types=[
        pltpu.VMEM((NCMAX, CHUNK), jnp.int32),
        pltpu.VMEM((NCMAX, CHUNK), jnp.int32),
        [pltpu.VMEM((CHUNK, D), _f32)] * NRING,
        [pltpu.SemaphoreType.DMA] * NRING,
        pltpu.VMEM_SHARED((NPAD, D), _f32),
    ],
    compiler_params=_sc_params,
)
def _sc_scatter(g_hbm, src_hbm, dst_hbm, out0, out1,
                src_v, dst_v, rows, gsem, acc):
    c = lax.axis_index("c")
    s = lax.axis_index("s")
    zeros16 = jnp.zeros((16,), _f32)

    def zrow(r, _):
        def zcol(k, _2):
            rows[0][r, pl.ds(k * 16, 16)] = zeros16
            return 0
        lax.fori_loop(0, D // 16, zcol, 0)
        return 0
    lax.fori_loop(0, CHUNK, zrow, 0)

    for k in range(RPT // CHUNK):
        pltpu.sync_copy(rows[0], acc.at[pl.ds(s * RPT + k * CHUNK, CHUNK)])
    rem = RPT % CHUNK
    pltpu.sync_copy(rows[0].at[pl.ds(0, rem)],
                    acc.at[pl.ds(s * RPT + (RPT // CHUNK) * CHUNK, rem)])

    def body(j, _):
        pltpu.async_copy(g_hbm.at[src_v.at[j]], rows[0], gsem[0]).wait()
        pltpu.sync_copy(rows[0], acc.at[dst_v.at[j]], add=True)
        return 0

    @pl.when(c == 0)
    def _():
        base = s * NCH0
        pltpu.sync_copy(src_hbm.at[pl.ds(base, NCH0)], src_v.at[pl.ds(0, NCH0)])
        pltpu.sync_copy(dst_hbm.at[pl.ds(base, NCH0)], dst_v.at[pl.ds(0, NCH0)])
        plsc.subcore_barrier()
        lax.fori_loop(0, NCH0, body, 0)

    @pl.when(c == 1)
    def _():
        base = NS * NCH0 + s * NCH1
        pltpu.sync_copy(src_hbm.at[pl.ds(base, NCH1)], src_v.at[pl.ds(0, NCH1)])
        pltpu.sync_copy(dst_hbm.at[pl.ds(base, NCH1)], dst_v.at[pl.ds(0, NCH1)])
        plsc.subcore_barrier()
        lax.fori_loop(0, NCH1, body, 0)

    plsc.subcore_barrier()

    @pl.when(c == 0)
    def _():
        pltpu.sync_copy(acc.at[pl.ds(s * RPT, RPT)], out0.at[pl.ds(s * RPT, RPT)])

    @pl.when(c == 1)
    def _():
        pltpu.sync_copy(acc.at[pl.ds(s * RPT, RPT)], out1.at[pl.ds(s * RPT, RPT)])


RB = 1000
_HI = jax.lax.Precision.HIGHEST


def _tc1_body(d0, d1, xr, w1, g1o, dvo):
    deg = 1.0 + d0[:, :1] + d1[:, :1]
    dv = jax.lax.rsqrt(deg)
    h = jnp.dot(xr[...], w1[...], preferred_element_type=_f32, precision=_HI)
    g1o[...] = h * dv
    dvo[...] = dv


def _tc2_body(a0, a1, g1r, dv, b1r, w2, g2o):
    z = (a0[...] + a1[...] + g1r[...]) * dv[...] + b1r[...]
    z = jnp.maximum(z, 0.0)
    g2o[...] = jnp.dot(z, w2[...], preferred_element_type=_f32,
                       precision=_HI) * dv[...]


def _tc3_body(a0, a1, g2r, dv, b2r, batch_r, out_ref, acc, cnt):
    i = pl.program_id(0)

    @pl.when(i == 0)
    def _():
        acc[...] = jnp.zeros_like(acc)
        cnt[...] = jnp.zeros_like(cnt)

    h = (a0[...] + a1[...] + g2r[...]) * dv[...] + b2r[...]
    bvec = batch_r[0]
    onehot = (lax.broadcasted_iota(jnp.int32, (B, RB), 0) == bvec).astype(_f32)
    acc[...] += jnp.dot(onehot, h, preferred_element_type=_f32, precision=_HI)
    cnt[...] += jnp.sum(onehot, axis=1, keepdims=True)

    @pl.when(i == pl.num_programs(0) - 1)
    def _():
        out_ref[...] = acc[...] / jnp.maximum(cnt[...], 1.0)


def _row_spec(w):
    return pl.BlockSpec((RB, w), lambda i: (i, 0))


def _const_spec(shape):
    return pl.BlockSpec(shape, lambda i: (0, 0))


def kernel(x, edge_index, batch, W1, b1, W2, b2):
    src = edge_index[0]
    dst = edge_index[1]
    pad = E_PAD - E
    src_p = jnp.concatenate([src, jnp.zeros((pad,), jnp.int32)])
    dst_p = jnp.concatenate([dst, jnp.full((pad,), N, jnp.int32)])
    src_f = src_p.reshape(TOTCH, CHUNK)
    dst_f = dst_p.reshape(TOTCH, CHUNK)
    dst_c = dst_p.reshape(NW, NCHUNK, CHUNK)
    b1r = b1.reshape(1, D)
    b2r = b2.reshape(1, D)
    batch_r = batch.reshape(N // RB, 1, RB)

    degp0, degp1 = _sc_degree(dst_c)

    g1, dinv = pl.pallas_call(
        _tc1_body,
        grid=(N // RB,),
        in_specs=[_row_spec(16), _row_spec(16), _row_spec(D), _const_spec((D, D))],
        out_specs=[_row_spec(D), _row_spec(1)],
        out_shape=[jax.ShapeDtypeStruct((N, D), _f32),
                   jax.ShapeDtypeStruct((N, 1), _f32)],
    )(degp0, degp1, x, W1)

    a0, a1 = _sc_scatter(g1, src_f, dst_f)

    g2 = pl.pallas_call(
        _tc2_body,
        grid=(N // RB,),
        in_specs=[_row_spec(D), _row_spec(D), _row_spec(D), _row_spec(1),
                  _const_spec((1, D)), _const_spec((D, D))],
        out_specs=_row_spec(D),
        out_shape=jax.ShapeDtypeStruct((N, D), _f32),
    )(a0, a1, g1, dinv, b1r, W2)

    c0, c1 = _sc_scatter(g2, src_f, dst_f)

    out = pl.pallas_call(
        _tc3_body,
        grid=(N // RB,),
        in_specs=[_row_spec(D), _row_spec(D), _row_spec(D), _row_spec(1),
                  _const_spec((1, D)),
                  pl.BlockSpec((1, 1, RB), lambda i: (i, 0, 0))],
        out_specs=_const_spec((B, D)),
        out_shape=jax.ShapeDtypeStruct((B, D), _f32),
        scratch_shapes=[pltpu.VMEM((B, D), _f32), pltpu.VMEM((B, D), _f32)],
    )(c0, c1, g2, dinv, b2r, batch_r)

    return out


# speedup vs baseline: 2.4856x; 1.0103x over previous
import functools

import jax
import jax.numpy as jnp
from jax import lax
from jax.experimental import pallas as pl
from jax.experimental.pallas import tpu as pltpu
from jax.experimental.pallas import tpu_sc as plsc

N = 10000
E = 320000
D = 128
B = 64

NC = 2
NS = 16
NW = NC * NS
CHUNK = 128
NCHUNK = 80
TOTCH = NW * NCHUNK
NCH0 = 128
NCH1 = 32
NCMAX = max(NCH0, NCH1)
NRING = 1
E_PAD = TOTCH * CHUNK
NPAD = 10112
RPT = NPAD // NS

_mesh = plsc.VectorSubcoreMesh(
    core_axis_name="c", subcore_axis_name="s", num_cores=NC, num_subcores=NS)

_sc_params = pltpu.CompilerParams(use_tc_tiling_on_sc=False)

_f32 = jnp.float32


@functools.partial(
    pl.kernel,
    out_type=[jax.ShapeDtypeStruct((NPAD, 16), _f32),
              jax.ShapeDtypeStruct((NPAD, 16), _f32)],
    mesh=_mesh,
    scratch_types=[
        pltpu.VMEM((NCHUNK, CHUNK), jnp.int32),
        pltpu.VMEM((CHUNK, 16), _f32),
        pltpu.VMEM((RPT, 16), _f32),
        pltpu.VMEM_SHARED((NPAD, 16), _f32),
    ],
    compiler_params=_sc_params,
)
def _sc_degree(dst_hbm, out0, out1, dst_v, ones_v, zb_v, acc):
    c = lax.axis_index("c")
    s = lax.axis_index("s")
    wid = s * NC + c
    ones16 = jnp.ones((16,), _f32)
    zeros16 = jnp.zeros((16,), _f32)

    def fill_ones(i, _):
        ones_v[i] = ones16
        return 0
    lax.fori_loop(0, CHUNK, fill_ones, 0)

    def fill_zeros(i, _):
        zb_v[i] = zeros16
        return 0
    lax.fori_loop(0, RPT, fill_zeros, 0)

    pltpu.sync_copy(zb_v, acc.at[pl.ds(s * RPT, RPT)])
    pltpu.sync_copy(dst_hbm.at[wid], dst_v)
    plsc.subcore_barrier()

    def body(j, _):
        pltpu.sync_copy(ones_v, acc.at[dst_v.at[j]], add=True)
        return 0
    lax.fori_loop(0, NCHUNK, body, 0)

    plsc.subcore_barrier()

    @pl.when(c == 0)
    def _():
        pltpu.sync_copy(acc.at[pl.ds(s * RPT, RPT)], out0.at[pl.ds(s * RPT, RPT)])

    @pl.when(c == 1)
    def _():
        pltpu.sync_copy(acc.at[pl.ds(s * RPT, RPT)], out1.at[pl.ds(s * RPT, RPT)])


@functools.partial(
    pl.kernel,
    out_type=[jax.ShapeDtypeStruct((NPAD, D), _f32),
              jax.ShapeDtypeStruct((NPAD, D), _f32)],
    mesh=_mesh,
    scratch_types=[
        pltpu.VMEM((NCMAX, CHUNK), jnp.int32),
        pltpu.VMEM((NCMAX, CHUNK), jnp.int32),
        [pltpu.VMEM((CHUNK, D), _f32)] * NRING,
        [pltpu.SemaphoreType.DMA] * NRING,
        pltpu.VMEM_SHARED((NPAD, D), _f32),
    ],
    compiler_params=_sc_params,
)
def _sc_scatter(g_hbm, src_hbm, dst_hbm, out0, out1,
                src_v, dst_v, rows, gsem, acc):
    c = lax.axis_index("c")
    s = lax.axis_index("s")
    zeros16 = jnp.zeros((16,), _f32)

    def zrow(r, _):
        def zcol(k, _2):
            rows[0][r, pl.ds(k * 16, 16)] = zeros16
            return 0
        lax.fori_loop(0, D // 16, zcol, 0)
        return 0
    lax.fori_loop(0, CHUNK, zrow, 0)

    for k in range(RPT // CHUNK):
        pltpu.sync_copy(rows[0], acc.at[pl.ds(s * RPT + k * CHUNK, CHUNK)])
    rem = RPT % CHUNK
    pltpu.sync_copy(rows[0].at[pl.ds(0, rem)],
                    acc.at[pl.ds(s * RPT + (RPT // CHUNK) * CHUNK, rem)])

    def body(j, _):
        pltpu.async_copy(g_hbm.at[src_v.at[j]], rows[0], gsem[0]).wait()
        pltpu.sync_copy(rows[0], acc.at[dst_v.at[j]], add=True)
        return 0

    @pl.when(c == 0)
    def _():
        base = s * NCH0
        pltpu.sync_copy(src_hbm.at[pl.ds(base, NCH0)], src_v.at[pl.ds(0, NCH0)])
        pltpu.sync_copy(dst_hbm.at[pl.ds(base, NCH0)], dst_v.at[pl.ds(0, NCH0)])
        plsc.subcore_barrier()
        lax.fori_loop(0, NCH0, body, 0)

    @pl.when(c == 1)
    def _():
        base = NS * NCH0 + s * NCH1
        pltpu.sync_copy(src_hbm.at[pl.ds(base, NCH1)], src_v.at[pl.ds(0, NCH1)])
        pltpu.sync_copy(dst_hbm.at[pl.ds(base, NCH1)], dst_v.at[pl.ds(0, NCH1)])
        plsc.subcore_barrier()
        lax.fori_loop(0, NCH1, body, 0)

    plsc.subcore_barrier()

    @pl.when(c == 0)
    def _():
        pltpu.sync_copy(acc.at[pl.ds(s * RPT, RPT)], out0.at[pl.ds(s * RPT, RPT)])

    @pl.when(c == 1)
    def _():
        pltpu.sync_copy(acc.at[pl.ds(s * RPT, RPT)], out1.at[pl.ds(s * RPT, RPT)])


RB = 1000
_HI = jax.lax.Precision.HIGHEST


def _tc1a_body(xr, w1, ho):
    ho[...] = jnp.dot(xr[...], w1[...], preferred_element_type=_f32,
                      precision=_HI)


def _tc1b_body(d0, d1, hr, g1o, dvo):
    deg = 1.0 + d0[:, :1] + d1[:, :1]
    dv = jax.lax.rsqrt(deg)
    g1o[...] = hr[...] * dv
    dvo[...] = dv


def _tc2_body(a0, a1, g1r, dv, b1r, w2, g2o):
    z = (a0[...] + a1[...] + g1r[...]) * dv[...] + b1r[...]
    z = jnp.maximum(z, 0.0)
    g2o[...] = jnp.dot(z, w2[...], preferred_element_type=_f32,
                       precision=_HI) * dv[...]


def _tc3_body(a0, a1, g2r, dv, b2r, batch_r, out_ref, acc, cnt):
    i = pl.program_id(0)

    @pl.when(i == 0)
    def _():
        acc[...] = jnp.zeros_like(acc)
        cnt[...] = jnp.zeros_like(cnt)

    h = (a0[...] + a1[...] + g2r[...]) * dv[...] + b2r[...]
    bvec = batch_r[0]
    onehot = (lax.broadcasted_iota(jnp.int32, (B, RB), 0) == bvec).astype(_f32)
    acc[...] += jnp.dot(onehot, h, preferred_element_type=_f32, precision=_HI)
    cnt[...] += jnp.sum(onehot, axis=1, keepdims=True)

    @pl.when(i == pl.num_programs(0) - 1)
    def _():
        out_ref[...] = acc[...] / jnp.maximum(cnt[...], 1.0)


def _row_spec(w):
    return pl.BlockSpec((RB, w), lambda i: (i, 0))


def _const_spec(shape):
    return pl.BlockSpec(shape, lambda i: (0, 0))


def kernel(x, edge_index, batch, W1, b1, W2, b2):
    src = edge_index[0]
    dst = edge_index[1]
    pad = E_PAD - E
    src_p = jnp.concatenate([src, jnp.zeros((pad,), jnp.int32)])
    dst_p = jnp.concatenate([dst, jnp.full((pad,), N, jnp.int32)])
    src_f = src_p.reshape(TOTCH, CHUNK)
    dst_f = dst_p.reshape(TOTCH, CHUNK)
    dst_c = dst_p.reshape(NW, NCHUNK, CHUNK)
    b1r = b1.reshape(1, D)
    b2r = b2.reshape(1, D)
    batch_r = batch.reshape(N // RB, 1, RB)

    degp0, degp1 = _sc_degree(dst_c)

    h1 = pl.pallas_call(
        _tc1a_body,
        grid=(N // RB,),
        in_specs=[_row_spec(D), _const_spec((D, D))],
        out_specs=_row_spec(D),
        out_shape=jax.ShapeDtypeStruct((N, D), _f32),
    )(x, W1)

    g1, dinv = pl.pallas_call(
        _tc1b_body,
        grid=(N // RB,),
        in_specs=[_row_spec(16), _row_spec(16), _row_spec(D)],
        out_specs=[_row_spec(D), _row_spec(1)],
        out_shape=[jax.ShapeDtypeStruct((N, D), _f32),
                   jax.ShapeDtypeStruct((N, 1), _f32)],
    )(degp0, degp1, h1)

    a0, a1 = _sc_scatter(g1, src_f, dst_f)

    g2 = pl.pallas_call(
        _tc2_body,
        grid=(N // RB,),
        in_specs=[_row_spec(D), _row_spec(D), _row_spec(D), _row_spec(1),
                  _const_spec((1, D)), _const_spec((D, D))],
        out_specs=_row_spec(D),
        out_shape=jax.ShapeDtypeStruct((N, D), _f32),
    )(a0, a1, g1, dinv, b1r, W2)

    c0, c1 = _sc_scatter(g2, src_f, dst_f)

    out = pl.pallas_call(
        _tc3_body,
        grid=(N // RB,),
        in_specs=[_row_spec(D), _row_spec(D), _row_spec(D), _row_spec(1),
                  _const_spec((1, D)),
                  pl.BlockSpec((1, 1, RB), lambda i: (i, 0, 0))],
        out_specs=_const_spec((B, D)),
        out_shape=jax.ShapeDtypeStruct((B, D), _f32),
        scratch_shapes=[pltpu.VMEM((B, D), _f32), pltpu.VMEM((B, D), _f32)],
    )(c0, c1, g2, dinv, b2r, batch_r)

    return out
